# Initial kernel scaffold; baseline (speedup 1.0000x reference)
import jax
import jax.numpy as jnp
from jax.experimental import pallas as pl


def kernel(center_word_indices, context_word_indices, negative_word_indices, center_table, context_table):
    raise NotImplementedError("write your pallas kernel here")


# speedup vs baseline: 5.7479x; 5.7479x over previous
import functools

import jax
import jax.numpy as jnp
from jax import lax
from jax.experimental import pallas as pl
from jax.experimental.pallas import tpu as pltpu
from jax.experimental.pallas import tpu_sc as plsc

B = 16384
D = 128
K = 20
KP1 = K + 1
NW = 32
BPW = B // NW
CH = 16
NCHUNK = BPW // CH
ROWS_PER_CHUNK = CH * KP1


def _sc_scores_kernel(cenidx_hbm, combidx_hbm, cen_tab_hbm, ctx_tab_hbm,
                      pos_hbm, neg_hbm,
                      cen_idx_v, comb_idx_v,
                      cen_a, cen_b, rows_a, rows_b,
                      pos_s, neg_s, sem_a, sem_b):
  wid = lax.axis_index("s") * 2 + lax.axis_index("c")
  base = wid * BPW

  pltpu.sync_copy(cenidx_hbm.at[pl.ds(base, BPW)], cen_idx_v)
  pltpu.sync_copy(combidx_hbm.at[pl.ds(base * KP1, BPW * KP1)], comb_idx_v)

  def gather_descs(g, cen_buf, rows_buf, sem):
    descs = [
        pltpu.make_async_copy(
            cen_tab_hbm.at[cen_idx_v.at[pl.ds(g * CH, CH)]], cen_buf, sem),
        pltpu.make_async_copy(
            ctx_tab_hbm.at[comb_idx_v.at[pl.ds(g * ROWS_PER_CHUNK, 128)]],
            rows_buf.at[pl.ds(0, 128)], sem),
        pltpu.make_async_copy(
            ctx_tab_hbm.at[comb_idx_v.at[pl.ds(g * ROWS_PER_CHUNK + 128, 128)]],
            rows_buf.at[pl.ds(128, 128)], sem),
        pltpu.make_async_copy(
            ctx_tab_hbm.at[comb_idx_v.at[pl.ds(g * ROWS_PER_CHUNK + 256, 80)]],
            rows_buf.at[pl.ds(256, 80)], sem),
    ]
    return descs

  def issue(g, cen_buf, rows_buf, sem):
    for d in gather_descs(g, cen_buf, rows_buf, sem):
      d.start()

  def drain(g, cen_buf, rows_buf, sem):
    for d in gather_descs(g, cen_buf, rows_buf, sem):
      d.wait()

  lane = lax.broadcasted_iota(jnp.int32, (16,), 0)
  last_lane = lane == 15

  def compute(g, cen_buf, rows_buf):
    def b_body(b, carry):
      c = [cen_buf[b, pl.ds(16 * j, 16)] for j in range(8)]
      gb = g * CH + b
      for j2 in range(KP1):
        r = b * KP1 + j2
        acc = c[0] * rows_buf[r, pl.ds(0, 16)]
        for j in range(1, 8):
          acc = acc + c[j] * rows_buf[r, pl.ds(16 * j, 16)]
        s = plsc.cumsum(acc)
        if j2 == 0:
          tgt = jnp.full((16,), gb, jnp.int32)
          plsc.store_scatter(pos_s, [tgt], s, mask=last_lane)
        else:
          tgt = jnp.full((16,), gb * K + (j2 - 1), jnp.int32)
          plsc.store_scatter(neg_s, [tgt], s, mask=last_lane)
      return carry
    lax.fori_loop(0, CH, b_body, 0)

  issue(0, cen_a, rows_a, sem_a)

  def body2(i, carry):
    g = 2 * i
    issue(g + 1, cen_b, rows_b, sem_b)
    drain(g, cen_a, rows_a, sem_a)
    compute(g, cen_a, rows_a)

    @pl.when(g + 2 < NCHUNK)
    def _():
      issue(g + 2, cen_a, rows_a, sem_a)

    drain(g + 1, cen_b, rows_b, sem_b)
    compute(g + 1, cen_b, rows_b)
    return carry

  lax.fori_loop(0, NCHUNK // 2, body2, 0)

  pltpu.sync_copy(pos_s, pos_hbm.at[pl.ds(base, BPW)])
  pltpu.sync_copy(neg_s, neg_hbm.at[pl.ds(base * K, BPW * K)])


@functools.partial(jax.jit, static_argnums=())
def _sc_scores(cen_idx, comb_idx, cen_tab, ctx_tab):
  mesh = plsc.VectorSubcoreMesh(core_axis_name="c", subcore_axis_name="s")
  f = pl.kernel(
      _sc_scores_kernel,
      out_type=[
          jax.ShapeDtypeStruct((B,), jnp.float32),
          jax.ShapeDtypeStruct((B * K,), jnp.float32),
      ],
      mesh=mesh,
      scratch_types=[
          pltpu.VMEM((BPW,), jnp.int32),
          pltpu.VMEM((BPW * KP1,), jnp.int32),
          pltpu.VMEM((CH, D), jnp.float32),
          pltpu.VMEM((CH, D), jnp.float32),
          pltpu.VMEM((ROWS_PER_CHUNK, D), jnp.float32),
          pltpu.VMEM((ROWS_PER_CHUNK, D), jnp.float32),
          pltpu.VMEM((BPW,), jnp.float32),
          pltpu.VMEM((BPW * K,), jnp.float32),
          pltpu.SemaphoreType.DMA,
          pltpu.SemaphoreType.DMA,
      ],
      compiler_params=pltpu.CompilerParams(needs_layout_passes=False),
  )
  return f(cen_idx, comb_idx, cen_tab, ctx_tab)


def _loss_body(pos_ref, neg_ref, out_ref):
  pos = pos_ref[...]
  neg = neg_ref[...]
  positive_loss = jnp.mean(jnp.logaddexp(-pos, 0.0))
  negative_loss = jnp.mean(jnp.logaddexp(neg, 0.0))
  out_ref[...] = jnp.reshape(positive_loss + negative_loss, (1, 1))


def _loss(pos2d, neg2d):
  return pl.pallas_call(
      _loss_body,
      out_shape=jax.ShapeDtypeStruct((1, 1), jnp.float32),
  )(pos2d, neg2d)


def kernel(center_word_indices, context_word_indices, negative_word_indices,
           center_table, context_table):
  cen_idx = center_word_indices.astype(jnp.int32)
  ctx_idx = context_word_indices.astype(jnp.int32)
  neg_idx = negative_word_indices.astype(jnp.int32)
  comb_idx = jnp.concatenate([ctx_idx[:, None], neg_idx], axis=1).reshape(-1)
  pos, neg = _sc_scores(cen_idx, comb_idx, center_table, context_table)
  loss = _loss(pos.reshape(B // D, D), neg.reshape(B * K // D, D))
  return loss[0, 0]


# speedup vs baseline: 12.3764x; 2.1532x over previous
import functools

import numpy as np
import jax
import jax.numpy as jnp
from jax import lax
from jax.experimental import pallas as pl
from jax.experimental.pallas import tpu as pltpu
from jax.experimental.pallas import tpu_sc as plsc

B = 16384
D = 128
K = 20
KP1 = K + 1
NW = 32
BPW = B // NW
CH = 16
NCHUNK = BPW // CH
ROWS_PER_CHUNK = CH * KP1
SPW = BPW * KP1


def _sc_scores_kernel(cenidx_hbm, combidx_hbm, cen_tab_hbm, ctx_tab_hbm,
                      sc_hbm,
                      cen_idx_v, comb_idx_v,
                      cen_a, cen_b, rows_a, rows_b,
                      sc_buf, sem_a, sem_b):
  wid = lax.axis_index("s") * 2 + lax.axis_index("c")
  base = wid * BPW

  pltpu.sync_copy(cenidx_hbm.at[pl.ds(base, BPW)], cen_idx_v)
  pltpu.sync_copy(combidx_hbm.at[pl.ds(base * KP1, SPW)], comb_idx_v)

  def gather_descs(g, cen_buf, rows_buf, sem):
    descs = [
        pltpu.make_async_copy(
            cen_tab_hbm.at[cen_idx_v.at[pl.ds(g * CH, CH)]], cen_buf, sem),
        pltpu.make_async_copy(
            ctx_tab_hbm.at[comb_idx_v.at[pl.ds(g * ROWS_PER_CHUNK, 128)]],
            rows_buf.at[pl.ds(0, 128)], sem),
        pltpu.make_async_copy(
            ctx_tab_hbm.at[comb_idx_v.at[pl.ds(g * ROWS_PER_CHUNK + 128, 128)]],
            rows_buf.at[pl.ds(128, 128)], sem),
        pltpu.make_async_copy(
            ctx_tab_hbm.at[comb_idx_v.at[pl.ds(g * ROWS_PER_CHUNK + 256, 80)]],
            rows_buf.at[pl.ds(256, 80)], sem),
    ]
    return descs

  def issue(g, cen_buf, rows_buf, sem):
    for d in gather_descs(g, cen_buf, rows_buf, sem):
      d.start()

  def drain(g, cen_buf, rows_buf, sem):
    for d in gather_descs(g, cen_buf, rows_buf, sem):
      d.wait()

  lane = lax.broadcasted_iota(jnp.int32, (16,), 0)
  shifts = [(lane + s) & 15 for s in (8, 4, 2, 1)]

  def compute(g, cen_buf, rows_buf):
    def b_body(b, carry):
      c = [cen_buf[b, pl.ds(16 * j, 16)] for j in range(8)]
      sbase = (g * CH + b) * KP1
      coll_a = coll_b = None
      for j2 in range(KP1):
        r = b * KP1 + j2
        p = [c[j] * rows_buf[r, pl.ds(16 * j, 16)] for j in range(8)]
        q = [p[0] + p[1], p[2] + p[3], p[4] + p[5], p[6] + p[7]]
        acc = (q[0] + q[1]) + (q[2] + q[3])
        for sh in shifts:
          acc = acc + acc.at[sh].get(mode="promise_in_bounds")
        if j2 == 0:
          coll_a = acc
        elif j2 < 16:
          coll_a = jnp.where(lane == j2, acc, coll_a)
        elif j2 == 16:
          coll_b = acc
        else:
          coll_b = jnp.where(lane == (j2 - 16), acc, coll_b)
      sc_buf[pl.ds(sbase, 16)] = coll_a
      sc_buf[pl.ds(sbase + 16, 16)] = coll_b
      return carry
    lax.fori_loop(0, CH, b_body, 0)

  issue(0, cen_a, rows_a, sem_a)

  def body2(i, carry):
    g = 2 * i
    issue(g + 1, cen_b, rows_b, sem_b)
    drain(g, cen_a, rows_a, sem_a)
    compute(g, cen_a, rows_a)

    @pl.when(g + 2 < NCHUNK)
    def _():
      issue(g + 2, cen_a, rows_a, sem_a)

    drain(g + 1, cen_b, rows_b, sem_b)
    compute(g + 1, cen_b, rows_b)
    return carry

  lax.fori_loop(0, NCHUNK // 2, body2, 0)

  pltpu.sync_copy(sc_buf.at[pl.ds(0, SPW)], sc_hbm.at[pl.ds(base * KP1, SPW)])


def _sc_scores(cen_idx, comb_idx, cen_tab, ctx_tab):
  mesh = plsc.VectorSubcoreMesh(core_axis_name="c", subcore_axis_name="s")
  f = pl.kernel(
      _sc_scores_kernel,
      out_type=jax.ShapeDtypeStruct((B * KP1,), jnp.float32),
      mesh=mesh,
      scratch_types=[
          pltpu.VMEM((BPW,), jnp.int32),
          pltpu.VMEM((SPW,), jnp.int32),
          pltpu.VMEM((CH, D), jnp.float32),
          pltpu.VMEM((CH, D), jnp.float32),
          pltpu.VMEM((ROWS_PER_CHUNK, D), jnp.float32),
          pltpu.VMEM((ROWS_PER_CHUNK, D), jnp.float32),
          pltpu.VMEM((SPW + 16,), jnp.float32),
          pltpu.SemaphoreType.DMA,
          pltpu.SemaphoreType.DMA,
      ],
      compiler_params=pltpu.CompilerParams(needs_layout_passes=False),
  )
  return f(cen_idx, comb_idx, cen_tab, ctx_tab)


def _loss_body(sc_ref, out_ref):
  x = sc_ref[...]
  rows, cols = x.shape
  gidx = (lax.broadcasted_iota(jnp.int32, x.shape, 0) * cols
          + lax.broadcasted_iota(jnp.int32, x.shape, 1))
  is_pos = (gidx % KP1) == 0
  contrib = jnp.where(is_pos,
                      jnp.logaddexp(-x, 0.0) * (1.0 / B),
                      jnp.logaddexp(x, 0.0) * (1.0 / (B * K)))
  out_ref[...] = jnp.reshape(jnp.sum(contrib), (1, 1))


def _loss(sc2d):
  return pl.pallas_call(
      _loss_body,
      out_shape=jax.ShapeDtypeStruct((1, 1), jnp.float32),
  )(sc2d)


def kernel(center_word_indices, context_word_indices, negative_word_indices,
           center_table, context_table):
  cen_idx = center_word_indices.astype(jnp.int32)
  ctx_idx = context_word_indices.astype(jnp.int32)
  neg_idx = negative_word_indices.astype(jnp.int32)
  comb_idx = jnp.concatenate([ctx_idx[:, None], neg_idx], axis=1).reshape(-1)
  scores = _sc_scores(cen_idx, comb_idx, center_table, context_table)
  loss = _loss(scores.reshape(B * KP1 // D, D))
  return loss[0, 0]
